# Initial kernel scaffold; baseline (speedup 1.0000x reference)
import jax
import jax.numpy as jnp
from jax.experimental import pallas as pl


def kernel(x, W_enc, W_dec, k):
    raise NotImplementedError("write your pallas kernel here")


# speedup vs baseline: 26.9528x; 26.9528x over previous
import functools

import jax
import jax.numpy as jnp
from jax import lax
from jax.experimental import pallas as pl
from jax.experimental.pallas import tpu as pltpu

D_MODEL = 1024
D_FF = 4096
TOKENS = 2 * 4096

_C = 8
_P = 10
_NB = 8
_INF_BITS = 0x7F800000


def _select_body(k_ref, we_ref, wd_ref, out_ref, state, cand, counts):
    p = pl.program_id(0)
    i = pl.program_id(1)
    k = k_ref[0]

    @pl.when(jnp.logical_and(p == 0, i == 0))
    def _init():
        state[0] = 0
        state[1] = _INF_BITS
        step = (_INF_BITS + _C) // (_C + 1)
        for j in range(_C):
            cand[j] = jnp.int32(min((j + 1) * step, _INF_BITS))
            counts[j] = 0

    be = lax.bitcast_convert_type(jnp.abs(we_ref[...]), jnp.int32)
    bd = lax.bitcast_convert_type(jnp.abs(wd_ref[...]), jnp.int32)
    for j in range(_C):
        c = cand[j]
        cnt = (jnp.sum((be >= c).astype(jnp.int32))
               + jnp.sum((bd >= c).astype(jnp.int32)))
        counts[j] = counts[j] + cnt

    @pl.when(i == _NB - 1)
    def _finalize():
        lo = state[0]
        hi = state[1]
        for j in range(_C):
            ge = counts[j] >= k
            lo = jnp.where(ge, jnp.maximum(lo, cand[j]), lo)
            hi = jnp.where(ge, hi, jnp.minimum(hi, cand[j]))
        state[0] = lo
        state[1] = hi
        step = (hi - lo + _C) // (_C + 1)
        for j in range(_C):
            cand[j] = jnp.minimum(lo + (j + 1) * step, hi)
            counts[j] = 0
        out_ref[...] = jnp.full((8, 128), lo, jnp.int32)


def _select(k_arr, W_enc, W_dec, interpret=False):
    grid_spec = pltpu.PrefetchScalarGridSpec(
        num_scalar_prefetch=1,
        grid=(_P, _NB),
        in_specs=[
            pl.BlockSpec((D_MODEL // _NB, D_FF), lambda p, i, k: (i, 0)),
            pl.BlockSpec((D_FF // _NB, D_MODEL), lambda p, i, k: (i, 0)),
        ],
        out_specs=pl.BlockSpec((8, 128), lambda p, i, k: (0, 0)),
        scratch_shapes=[
            pltpu.SMEM((2,), jnp.int32),
            pltpu.SMEM((_C,), jnp.int32),
            pltpu.SMEM((_C,), jnp.int32),
        ],
    )
    return pl.pallas_call(
        _select_body,
        grid_spec=grid_spec,
        out_shape=jax.ShapeDtypeStruct((8, 128), jnp.int32),
        interpret=interpret,
    )(k_arr, W_enc, W_dec)


def _mask_body(t_ref, we_ref, wd_ref, weo_ref, wdo_ref):
    t = t_ref[0]
    we = we_ref[...]
    wd = wd_ref[...]
    weo_ref[...] = jnp.where(jnp.abs(we) >= t, we, 0.0).astype(jnp.bfloat16)
    wdo_ref[...] = jnp.where(jnp.abs(wd) >= t, wd, 0.0).astype(jnp.bfloat16)


def _mask(t_arr, W_enc, W_dec, interpret=False):
    nb = 8
    grid_spec = pltpu.PrefetchScalarGridSpec(
        num_scalar_prefetch=1,
        grid=(nb,),
        in_specs=[
            pl.BlockSpec((D_MODEL // nb, D_FF), lambda i, t: (i, 0)),
            pl.BlockSpec((D_FF // nb, D_MODEL), lambda i, t: (i, 0)),
        ],
        out_specs=[
            pl.BlockSpec((D_MODEL // nb, D_FF), lambda i, t: (i, 0)),
            pl.BlockSpec((D_FF // nb, D_MODEL), lambda i, t: (i, 0)),
        ],
    )
    return pl.pallas_call(
        _mask_body,
        grid_spec=grid_spec,
        out_shape=[
            jax.ShapeDtypeStruct((D_MODEL, D_FF), jnp.bfloat16),
            jax.ShapeDtypeStruct((D_FF, D_MODEL), jnp.bfloat16),
        ],
        interpret=interpret,
    )(t_arr, W_enc, W_dec)


_BT = 2048
_BF = 512


def _mlp_body(x_ref, we_ref, wd_ref, y_ref):
    f = pl.program_id(1)
    xb = x_ref[...].astype(jnp.bfloat16)
    h = jnp.dot(xb, we_ref[...], preferred_element_type=jnp.float32)
    h = jnp.maximum(h, 0.0).astype(jnp.bfloat16)
    yb = jnp.dot(h, wd_ref[...], preferred_element_type=jnp.float32)

    @pl.when(f == 0)
    def _first():
        y_ref[...] = yb

    @pl.when(f > 0)
    def _acc():
        y_ref[...] += yb


def _mlp(xf, We_b, Wd_b, interpret=False):
    grid = (TOKENS // _BT, D_FF // _BF)
    return pl.pallas_call(
        _mlp_body,
        grid=grid,
        in_specs=[
            pl.BlockSpec((_BT, D_MODEL), lambda t, f: (t, 0)),
            pl.BlockSpec((D_MODEL, _BF), lambda t, f: (0, f)),
            pl.BlockSpec((_BF, D_MODEL), lambda t, f: (f, 0)),
        ],
        out_specs=pl.BlockSpec((_BT, D_MODEL), lambda t, f: (t, 0)),
        out_shape=jax.ShapeDtypeStruct((TOKENS, D_MODEL), jnp.float32),
        interpret=interpret,
    )(xf, We_b, Wd_b)


def kernel(x, W_enc, W_dec, k):
    k_arr = jnp.asarray(k, jnp.int32).reshape(1)
    t_bits = _select(k_arr, W_enc, W_dec)
    t = lax.bitcast_convert_type(t_bits[0, 0], jnp.float32).reshape(1)
    We_b, Wd_b = _mask(t, W_enc, W_dec)
    xf = x.reshape(TOKENS, D_MODEL)
    y = _mlp(xf, We_b, Wd_b)
    return y.reshape(x.shape)
